# Initial kernel scaffold; baseline (speedup 1.0000x reference)
import jax
import jax.numpy as jnp
from jax.experimental import pallas as pl


def kernel(features, edge_index, W1, b1, W2, b2, W3, b3, W4, b4, W5, b5):
    raise NotImplementedError("write your pallas kernel here")


# speedup vs baseline: 7.6963x; 7.6963x over previous
import functools

import jax
import jax.numpy as jnp
from jax import lax
from jax.experimental import pallas as pl
from jax.experimental.pallas import tpu as pltpu
from jax.experimental.pallas import tpu_sc as plsc

NC = 2
NS = 16
SUB = 128
KSUB = 8
CHUNK = SUB * KSUB


def _leaky(x):
    return jnp.where(x >= 0, x, 0.01 * x)


def _mesh():
    return plsc.VectorSubcoreMesh(core_axis_name="c", subcore_axis_name="s",
                                  num_cores=NC, num_subcores=NS)


def _sc_degree(N_pad, E_pad):
    NpT = N_pad // NS
    per_tile = E_pad // NS
    n_chunks = per_tile // CHUNK
    rows_pt = per_tile // SUB
    rows_pc = E_pad // SUB

    @functools.partial(
        pl.kernel,
        out_type=jax.ShapeDtypeStruct((NC, N_pad, 1), jnp.float32),
        mesh=_mesh(),
        compiler_params=pltpu.CompilerParams(use_tc_tiling_on_sc=False),
        scratch_types=[
            pltpu.VMEM((KSUB, SUB), jnp.int32),
            pltpu.VMEM((SUB, 1), jnp.float32),
            pltpu.VMEM_SHARED((N_pad, 1), jnp.float32),
        ],
    )
    def kern(idx2d, ones_hbm, zeros_hbm, out_hbm, idxv, onesv, hist):
        c = lax.axis_index("c")
        t = lax.axis_index("s")
        pltpu.sync_copy(ones_hbm, onesv)
        pltpu.sync_copy(zeros_hbm, hist.at[pl.ds(t * NpT, NpT)])
        plsc.subcore_barrier()

        def step(i, carry):
            row0 = c * rows_pc + t * rows_pt + i * KSUB
            pltpu.sync_copy(idx2d.at[pl.ds(row0, KSUB)], idxv)
            for k in range(KSUB):
                pltpu.sync_copy(onesv, hist.at[idxv.at[k]], add=True)
            return carry

        lax.fori_loop(0, n_chunks, step, 0)
        plsc.subcore_barrier()

        @pl.when(c == 0)
        def _():
            pltpu.sync_copy(hist.at[pl.ds(t * NpT, NpT)],
                            out_hbm.at[0, pl.ds(t * NpT, NpT)])

        @pl.when(c == 1)
        def _():
            pltpu.sync_copy(hist.at[pl.ds(t * NpT, NpT)],
                            out_hbm.at[1, pl.ds(t * NpT, NpT)])

    return kern


def _sc_propagate_split(N_pad, E_pad, w):
    NpT = N_pad // NS
    per_tile = E_pad // (NC * NS)
    n_chunks = per_tile // CHUNK
    rows_pt = per_tile // SUB
    rows_pc = E_pad // NC // SUB

    @functools.partial(
        pl.kernel,
        out_type=jax.ShapeDtypeStruct((NC, N_pad, w), jnp.float32),
        mesh=_mesh(),
        compiler_params=pltpu.CompilerParams(use_tc_tiling_on_sc=False),
        scratch_types=[
            pltpu.VMEM((KSUB, SUB), jnp.int32),
            pltpu.VMEM((KSUB, SUB), jnp.int32),
            pltpu.VMEM((CHUNK, w), jnp.float32),
            pltpu.VMEM_SHARED((N_pad, w), jnp.float32),
            pltpu.SemaphoreType.DMA,
        ],
    )
    def kern(src2d, dst2d, h_hbm, zeros_hbm, out_hbm, idxs, idxd, rows, acc, sem):
        c = lax.axis_index("c")
        t = lax.axis_index("s")
        pltpu.sync_copy(zeros_hbm, acc.at[pl.ds(t * NpT, NpT)])
        plsc.subcore_barrier()

        def step(i, carry):
            row0 = c * rows_pc + t * rows_pt + i * KSUB
            pltpu.sync_copy(src2d.at[pl.ds(row0, KSUB)], idxs)
            pltpu.sync_copy(dst2d.at[pl.ds(row0, KSUB)], idxd)
            descs = [
                pltpu.async_copy(h_hbm.at[idxs.at[k]],
                                 rows.at[pl.ds(k * SUB, SUB)], sem)
                for k in range(KSUB)
            ]
            for d in descs:
                d.wait()
            for k in range(KSUB):
                pltpu.sync_copy(rows.at[pl.ds(k * SUB, SUB)],
                                acc.at[idxd.at[k]], add=True)
            return carry

        lax.fori_loop(0, n_chunks, step, 0)
        plsc.subcore_barrier()

        @pl.when(c == 0)
        def _():
            pltpu.sync_copy(acc.at[pl.ds(t * NpT, NpT)],
                            out_hbm.at[0, pl.ds(t * NpT, NpT)])

        @pl.when(c == 1)
        def _():
            pltpu.sync_copy(acc.at[pl.ds(t * NpT, NpT)],
                            out_hbm.at[1, pl.ds(t * NpT, NpT)])

    return kern


def _sc_propagate_sliced(N_pad, E_pad):
    NpT = N_pad // NS
    per_tile = E_pad // NS
    n_chunks = per_tile // CHUNK
    rows_pt = per_tile // SUB

    @functools.partial(
        pl.kernel,
        out_type=jax.ShapeDtypeStruct((4, N_pad, 16), jnp.float32),
        mesh=_mesh(),
        compiler_params=pltpu.CompilerParams(use_tc_tiling_on_sc=False),
        scratch_types=[
            pltpu.VMEM((KSUB, SUB), jnp.int32),
            pltpu.VMEM((KSUB, SUB), jnp.int32),
            pltpu.VMEM((CHUNK, 16), jnp.float32),
            pltpu.VMEM_SHARED((N_pad, 16), jnp.float32),
            pltpu.SemaphoreType.DMA,
        ],
    )
    def kern(src2d, dst2d, h_hbm, zeros_hbm, out_hbm, idxs, idxd, rows, acc, sem):
        c = lax.axis_index("c")
        t = lax.axis_index("s")

        def gather_all(h2d):
            descs = [
                pltpu.async_copy(h2d.at[idxs.at[k]],
                                 rows.at[pl.ds(k * SUB, SUB)], sem)
                for k in range(KSUB)
            ]
            for d in descs:
                d.wait()

        for j in range(2):
            pltpu.sync_copy(zeros_hbm, acc.at[pl.ds(t * NpT, NpT)])
            plsc.subcore_barrier()

            def step(i, carry):
                row0 = t * rows_pt + i * KSUB
                pltpu.sync_copy(src2d.at[pl.ds(row0, KSUB)], idxs)
                pltpu.sync_copy(dst2d.at[pl.ds(row0, KSUB)], idxd)

                @pl.when(c == 0)
                def _():
                    gather_all(h_hbm.at[j])

                @pl.when(c == 1)
                def _():
                    gather_all(h_hbm.at[2 + j])

                for k in range(KSUB):
                    pltpu.sync_copy(rows.at[pl.ds(k * SUB, SUB)],
                                    acc.at[idxd.at[k]], add=True)
                return carry

            lax.fori_loop(0, n_chunks, step, 0)
            plsc.subcore_barrier()

            @pl.when(c == 0)
            def _():
                pltpu.sync_copy(acc.at[pl.ds(t * NpT, NpT)],
                                out_hbm.at[j, pl.ds(t * NpT, NpT)])

            @pl.when(c == 1)
            def _():
                pltpu.sync_copy(acc.at[pl.ds(t * NpT, NpT)],
                                out_hbm.at[2 + j, pl.ds(t * NpT, NpT)])

    return kern


def _tc_norms(N, N_pad, Nb):
    def body(degs_ref, degd_ref, feat_ref, ns_ref, nd_ref, h1_ref):
        ns = lax.rsqrt(jnp.maximum(degs_ref[...], 1.0))
        nd = lax.rsqrt(jnp.maximum(degd_ref[...], 1.0))
        ns_ref[...] = ns
        nd_ref[...] = nd
        h1_ref[...] = feat_ref[...] * ns

    return pl.pallas_call(
        body,
        grid=(N // Nb,),
        in_specs=[
            pl.BlockSpec((Nb, 1), lambda i: (i, 0)),
            pl.BlockSpec((Nb, 1), lambda i: (i, 0)),
            pl.BlockSpec((Nb, 4), lambda i: (i, 0)),
        ],
        out_specs=[
            pl.BlockSpec((Nb, 1), lambda i: (i, 0)),
            pl.BlockSpec((Nb, 1), lambda i: (i, 0)),
            pl.BlockSpec((Nb, 4), lambda i: (i, 0)),
        ],
        out_shape=[
            jax.ShapeDtypeStruct((N, 1), jnp.float32),
            jax.ShapeDtypeStruct((N, 1), jnp.float32),
            jax.ShapeDtypeStruct((N_pad, 4), jnp.float32),
        ],
    )


def _tc_layer1(N, N_pad, Nb):
    def body(p_ref, w_ref, b_ref, nd_ref, ns_ref, o_ref):
        agg = p_ref[0] + p_ref[1]
        z = jnp.dot(agg, w_ref[...], preferred_element_type=jnp.float32)
        x = _leaky(z * nd_ref[...] + b_ref[...][None, :])
        hn = x * ns_ref[...]
        for k in range(4):
            o_ref[k] = hn[:, 16 * k:16 * (k + 1)]

    return pl.pallas_call(
        body,
        grid=(N // Nb,),
        in_specs=[
            pl.BlockSpec((2, Nb, 4), lambda i: (0, i, 0)),
            pl.BlockSpec((4, 64), lambda i: (0, 0)),
            pl.BlockSpec((64,), lambda i: (0,)),
            pl.BlockSpec((Nb, 1), lambda i: (i, 0)),
            pl.BlockSpec((Nb, 1), lambda i: (i, 0)),
        ],
        out_specs=pl.BlockSpec((4, Nb, 16), lambda i: (0, i, 0)),
        out_shape=jax.ShapeDtypeStruct((4, N_pad, 16), jnp.float32),
    )


def _tc_mid(N, N_pad, Nb, last):
    def body(a_ref, w_ref, b_ref, nd_ref, ns_ref, *rest):
        h = jnp.concatenate([a_ref[0], a_ref[1], a_ref[2], a_ref[3]], axis=1)
        z = jnp.dot(h, w_ref[...], preferred_element_type=jnp.float32)
        x = _leaky(z * nd_ref[...] + b_ref[...][None, :])
        hn = x * ns_ref[...]
        if last:
            w5_ref, o_ref = rest
            o_ref[...] = jnp.dot(hn, w5_ref[...],
                                 preferred_element_type=jnp.float32)
        else:
            (o_ref,) = rest
            for k in range(4):
                o_ref[k] = hn[:, 16 * k:16 * (k + 1)]

    in_specs = [
        pl.BlockSpec((4, Nb, 16), lambda i: (0, i, 0)),
        pl.BlockSpec((64, 64), lambda i: (0, 0)),
        pl.BlockSpec((64,), lambda i: (0,)),
        pl.BlockSpec((Nb, 1), lambda i: (i, 0)),
        pl.BlockSpec((Nb, 1), lambda i: (i, 0)),
    ]
    if last:
        in_specs.append(pl.BlockSpec((64, 3), lambda i: (0, 0)))
        out_specs = pl.BlockSpec((Nb, 3), lambda i: (i, 0))
        out_shape = jax.ShapeDtypeStruct((N_pad, 3), jnp.float32)
    else:
        out_specs = pl.BlockSpec((4, Nb, 16), lambda i: (0, i, 0))
        out_shape = jax.ShapeDtypeStruct((4, N_pad, 16), jnp.float32)
    return pl.pallas_call(
        body, grid=(N // Nb,), in_specs=in_specs, out_specs=out_specs,
        out_shape=out_shape)


def _tc_out(N, Nb):
    def body(p_ref, b_ref, nd_ref, o_ref):
        o_ref[...] = ((p_ref[0] + p_ref[1]) * nd_ref[...]
                      + b_ref[...][None, :])

    return pl.pallas_call(
        body,
        grid=(N // Nb,),
        in_specs=[
            pl.BlockSpec((2, Nb, 3), lambda i: (0, i, 0)),
            pl.BlockSpec((3,), lambda i: (0,)),
            pl.BlockSpec((Nb, 1), lambda i: (i, 0)),
        ],
        out_specs=pl.BlockSpec((Nb, 3), lambda i: (i, 0)),
        out_shape=jax.ShapeDtypeStruct((N, 3), jnp.float32),
    )


def kernel(features, edge_index, W1, b1, W2, b2, W3, b3, W4, b4, W5, b5):
    N = features.shape[0]
    E = edge_index.shape[1]
    assert N % 16 == 0
    N_pad = -(-(N + 1) // (NS * 8)) * (NS * 8)
    align = NC * NS * CHUNK
    E_pad = -(-E // align) * align
    Nb = 4000 if N % 4000 == 0 else 1000
    assert N % Nb == 0

    src = edge_index[0]
    dst = edge_index[1]
    if E_pad != E:
        fill = jnp.full((E_pad - E,), N, jnp.int32)
        src = jnp.concatenate([src, fill])
        dst = jnp.concatenate([dst, fill])
    src2d = src.reshape(E_pad // SUB, SUB)
    dst2d = dst.reshape(E_pad // SUB, SUB)
    idx2d = jnp.concatenate([src, dst]).reshape(2 * E_pad // SUB, SUB)

    ones = jnp.ones((SUB, 1), jnp.float32)
    NpT = N_pad // NS
    z1 = jnp.zeros((NpT, 1), jnp.float32)
    z4 = jnp.zeros((NpT, 4), jnp.float32)
    z16 = jnp.zeros((NpT, 16), jnp.float32)
    z3 = jnp.zeros((NpT, 3), jnp.float32)

    deg2 = _sc_degree(N_pad, E_pad)(idx2d, ones, z1)
    ns_, nd_, h1 = _tc_norms(N, N_pad, Nb)(
        deg2[0, :N], deg2[1, :N], features)

    p1 = _sc_propagate_split(N_pad, E_pad, 4)(src2d, dst2d, h1, z4)
    h = _tc_layer1(N, N_pad, Nb)(p1, W1, b1, nd_, ns_)

    prop64 = _sc_propagate_sliced(N_pad, E_pad)
    for W, b in ((W2, b2), (W3, b3)):
        agg = prop64(src2d, dst2d, h, z16)
        h = _tc_mid(N, N_pad, Nb, last=False)(agg, W, b, nd_, ns_)

    agg = prop64(src2d, dst2d, h, z16)
    t5 = _tc_mid(N, N_pad, Nb, last=True)(agg, W4, b4, nd_, ns_, W5)

    p5 = _sc_propagate_split(N_pad, E_pad, 3)(src2d, dst2d, t5, z3)
    return _tc_out(N, Nb)(p5, b5, nd_)


# speedup vs baseline: 8.1133x; 1.0542x over previous
import functools

import jax
import jax.numpy as jnp
from jax import lax
from jax.experimental import pallas as pl
from jax.experimental.pallas import tpu as pltpu
from jax.experimental.pallas import tpu_sc as plsc

NC = 2
NS = 16
SUB = 128
KSUB = 4
CHUNK = SUB * KSUB


def _leaky(x):
    return jnp.where(x >= 0, x, 0.01 * x)


def _mesh():
    return plsc.VectorSubcoreMesh(core_axis_name="c", subcore_axis_name="s",
                                  num_cores=NC, num_subcores=NS)


def _run_pipeline(n_chunks, load_idx, wait_idx, do_gathers, fire_scatters,
                  drain_scatters, sync_scatters=False):
    assert n_chunks >= 4 and n_chunks % 2 == 0

    def iter_body(i, b, first, last):
        wait_idx(b)
        do_gathers(b)
        if sync_scatters:
            fire_scatters(b)
        else:
            if not first:
                drain_scatters(1 - b)
            fire_scatters(b)
        if not last:
            load_idx(i + 1, 1 - b)

    load_idx(0, 0)
    iter_body(0, 0, True, False)

    def pair(p, carry):
        i = 1 + 2 * p
        iter_body(i, 1, False, False)
        iter_body(i + 1, 0, False, False)
        return carry

    lax.fori_loop(0, (n_chunks - 2) // 2, pair, 0)
    iter_body(n_chunks - 1, 1, False, True)
    if not sync_scatters:
        drain_scatters(1)


def _sc_degree(N_pad, E_pad):
    K8 = 8
    NpT = N_pad // NS
    per_tile = E_pad // NS
    n_chunks = per_tile // (SUB * K8)
    rows_pt = per_tile // SUB
    rows_pc = E_pad // SUB
    assert per_tile % (SUB * K8) == 0

    @functools.partial(
        pl.kernel,
        out_type=jax.ShapeDtypeStruct((NC, N_pad, 1), jnp.float32),
        mesh=_mesh(),
        compiler_params=pltpu.CompilerParams(use_tc_tiling_on_sc=False),
        scratch_types=[
            pltpu.VMEM((K8, SUB), jnp.int32),
            pltpu.VMEM((SUB, 1), jnp.float32),
            pltpu.VMEM_SHARED((N_pad, 1), jnp.float32),
        ],
    )
    def kern(idx2d, ones_hbm, zeros_hbm, out_hbm, idxv, onesv, hist):
        c = lax.axis_index("c")
        t = lax.axis_index("s")
        pltpu.sync_copy(ones_hbm, onesv)
        pltpu.sync_copy(zeros_hbm, hist.at[pl.ds(t * NpT, NpT)])
        plsc.subcore_barrier()

        def step(i, carry):
            row0 = c * rows_pc + t * rows_pt + i * K8
            pltpu.sync_copy(idx2d.at[pl.ds(row0, K8)], idxv)
            for k in range(K8):
                pltpu.sync_copy(onesv, hist.at[idxv.at[k]], add=True)
            return carry

        lax.fori_loop(0, n_chunks, step, 0)
        plsc.subcore_barrier()

        @pl.when(c == 0)
        def _():
            pltpu.sync_copy(hist.at[pl.ds(t * NpT, NpT)],
                            out_hbm.at[0, pl.ds(t * NpT, NpT)])

        @pl.when(c == 1)
        def _():
            pltpu.sync_copy(hist.at[pl.ds(t * NpT, NpT)],
                            out_hbm.at[1, pl.ds(t * NpT, NpT)])

    return kern


def _sc_propagate_split(N_pad, E_pad, w):
    K8 = 8
    NpT = N_pad // NS
    per_tile = E_pad // (NC * NS)
    n_chunks = per_tile // (SUB * K8)
    rows_pt = per_tile // SUB
    rows_pc = E_pad // NC // SUB
    assert per_tile % (SUB * K8) == 0

    @functools.partial(
        pl.kernel,
        out_type=jax.ShapeDtypeStruct((NC, N_pad, w), jnp.float32),
        mesh=_mesh(),
        compiler_params=pltpu.CompilerParams(use_tc_tiling_on_sc=False),
        scratch_types=[
            pltpu.VMEM((K8, SUB), jnp.int32),
            pltpu.VMEM((K8, SUB), jnp.int32),
            pltpu.VMEM((SUB * K8, w), jnp.float32),
            pltpu.VMEM_SHARED((N_pad, w), jnp.float32),
            pltpu.SemaphoreType.DMA,
        ],
    )
    def kern(src2d, dst2d, h_hbm, zeros_hbm, out_hbm, idxs, idxd, rows, acc,
             sem):
        c = lax.axis_index("c")
        t = lax.axis_index("s")
        pltpu.sync_copy(zeros_hbm, acc.at[pl.ds(t * NpT, NpT)])
        plsc.subcore_barrier()

        def step(i, carry):
            row0 = c * rows_pc + t * rows_pt + i * K8
            pltpu.sync_copy(src2d.at[pl.ds(row0, K8)], idxs)
            pltpu.sync_copy(dst2d.at[pl.ds(row0, K8)], idxd)
            descs = [
                pltpu.async_copy(h_hbm.at[idxs.at[k]],
                                 rows.at[pl.ds(k * SUB, SUB)], sem)
                for k in range(K8)
            ]
            for d in descs:
                d.wait()
            for k in range(K8):
                pltpu.sync_copy(rows.at[pl.ds(k * SUB, SUB)],
                                acc.at[idxd.at[k]], add=True)
            return carry

        lax.fori_loop(0, n_chunks, step, 0)
        plsc.subcore_barrier()

        @pl.when(c == 0)
        def _():
            pltpu.sync_copy(acc.at[pl.ds(t * NpT, NpT)],
                            out_hbm.at[0, pl.ds(t * NpT, NpT)])

        @pl.when(c == 1)
        def _():
            pltpu.sync_copy(acc.at[pl.ds(t * NpT, NpT)],
                            out_hbm.at[1, pl.ds(t * NpT, NpT)])

    return kern


def _sc_propagate_sliced(N_pad, E_pad):
    NpT = N_pad // NS
    per_tile = E_pad // NS
    n_chunks = per_tile // CHUNK
    rows_pt = per_tile // SUB

    @functools.partial(
        pl.kernel,
        out_type=jax.ShapeDtypeStruct((4, N_pad, 16), jnp.float32),
        mesh=_mesh(),
        compiler_params=pltpu.CompilerParams(use_tc_tiling_on_sc=False),
        scratch_types=[
            pltpu.VMEM((2, KSUB, SUB), jnp.int32),
            pltpu.VMEM((2, KSUB, SUB), jnp.int32),
            pltpu.VMEM((2, CHUNK, 16), jnp.float32),
            pltpu.VMEM_SHARED((N_pad, 16), jnp.float32),
            pltpu.SemaphoreType.DMA,
            pltpu.SemaphoreType.DMA,
            pltpu.SemaphoreType.DMA,
            pltpu.SemaphoreType.DMA,
            pltpu.SemaphoreType.DMA,
            pltpu.SemaphoreType.DMA,
        ],
    )
    def kern(src2d, dst2d, h_hbm, zeros_hbm, out_hbm, idxs, idxd, rows, acc,
             si0, si1, sg0, sg1, ss0, ss1):
        c = lax.axis_index("c")
        t = lax.axis_index("s")
        sem_i = (si0, si1)
        sem_g = (sg0, sg1)
        sem_s = (ss0, ss1)

        def load_idx(i, b):
            row0 = t * rows_pt + i * KSUB
            pltpu.async_copy(src2d.at[pl.ds(row0, KSUB)], idxs.at[b], sem_i[b])
            pltpu.async_copy(dst2d.at[pl.ds(row0, KSUB)], idxd.at[b], sem_i[b])

        def wait_idx(b):
            pltpu.make_async_copy(src2d.at[pl.ds(0, KSUB)], idxs.at[b],
                                  sem_i[b]).wait()
            pltpu.make_async_copy(dst2d.at[pl.ds(0, KSUB)], idxd.at[b],
                                  sem_i[b]).wait()

        def fire_scatters(b):
            for k in range(KSUB):
                pltpu.async_copy(rows.at[b].at[pl.ds(k * SUB, SUB)],
                                 acc.at[idxd.at[b].at[k]], sem_s[b], add=True)

        def drain_scatters(b):
            for k in range(KSUB):
                pltpu.make_async_copy(rows.at[b].at[pl.ds(k * SUB, SUB)],
                                      acc.at[idxd.at[b].at[k]],
                                      sem_s[b]).wait()

        for j in range(2):

            def _fire_gathers(h2d, b):
                return [
                    pltpu.async_copy(h2d.at[idxs.at[b].at[k]],
                                     rows.at[b].at[pl.ds(k * SUB, SUB)],
                                     sem_g[b])
                    for k in range(KSUB)
                ]

            def do_gathers(b, j=j):
                @pl.when(c == 0)
                def _():
                    for d in _fire_gathers(h_hbm.at[j], b):
                        d.wait()

                @pl.when(c == 1)
                def _():
                    for d in _fire_gathers(h_hbm.at[2 + j], b):
                        d.wait()

            pltpu.sync_copy(zeros_hbm, acc.at[pl.ds(t * NpT, NpT)])
            plsc.subcore_barrier()
            _run_pipeline(n_chunks, load_idx, wait_idx, do_gathers,
                          fire_scatters, drain_scatters)
            plsc.subcore_barrier()

            @pl.when(c == 0)
            def _(j=j):
                pltpu.sync_copy(acc.at[pl.ds(t * NpT, NpT)],
                                out_hbm.at[j, pl.ds(t * NpT, NpT)])

            @pl.when(c == 1)
            def _(j=j):
                pltpu.sync_copy(acc.at[pl.ds(t * NpT, NpT)],
                                out_hbm.at[2 + j, pl.ds(t * NpT, NpT)])

    return kern


def _tc_norms(N, N_pad, Nb):
    def body(degs_ref, degd_ref, feat_ref, ns_ref, nd_ref, h1_ref):
        ns = lax.rsqrt(jnp.maximum(degs_ref[...], 1.0))
        nd = lax.rsqrt(jnp.maximum(degd_ref[...], 1.0))
        ns_ref[...] = ns
        nd_ref[...] = nd
        h1_ref[...] = feat_ref[...] * ns

    return pl.pallas_call(
        body,
        grid=(N // Nb,),
        in_specs=[
            pl.BlockSpec((Nb, 1), lambda i: (i, 0)),
            pl.BlockSpec((Nb, 1), lambda i: (i, 0)),
            pl.BlockSpec((Nb, 4), lambda i: (i, 0)),
        ],
        out_specs=[
            pl.BlockSpec((Nb, 1), lambda i: (i, 0)),
            pl.BlockSpec((Nb, 1), lambda i: (i, 0)),
            pl.BlockSpec((Nb, 4), lambda i: (i, 0)),
        ],
        out_shape=[
            jax.ShapeDtypeStruct((N, 1), jnp.float32),
            jax.ShapeDtypeStruct((N, 1), jnp.float32),
            jax.ShapeDtypeStruct((N_pad, 4), jnp.float32),
        ],
    )


def _tc_layer1(N, N_pad, Nb):
    def body(p_ref, w_ref, b_ref, nd_ref, ns_ref, o_ref):
        agg = p_ref[0] + p_ref[1]
        z = jnp.dot(agg, w_ref[...], preferred_element_type=jnp.float32)
        x = _leaky(z * nd_ref[...] + b_ref[...][None, :])
        hn = x * ns_ref[...]
        for k in range(4):
            o_ref[k] = hn[:, 16 * k:16 * (k + 1)]

    return pl.pallas_call(
        body,
        grid=(N // Nb,),
        in_specs=[
            pl.BlockSpec((2, Nb, 4), lambda i: (0, i, 0)),
            pl.BlockSpec((4, 64), lambda i: (0, 0)),
            pl.BlockSpec((64,), lambda i: (0,)),
            pl.BlockSpec((Nb, 1), lambda i: (i, 0)),
            pl.BlockSpec((Nb, 1), lambda i: (i, 0)),
        ],
        out_specs=pl.BlockSpec((4, Nb, 16), lambda i: (0, i, 0)),
        out_shape=jax.ShapeDtypeStruct((4, N_pad, 16), jnp.float32),
    )


def _tc_mid(N, N_pad, Nb, last):
    def body(a_ref, w_ref, b_ref, nd_ref, ns_ref, *rest):
        h = jnp.concatenate([a_ref[0], a_ref[1], a_ref[2], a_ref[3]], axis=1)
        z = jnp.dot(h, w_ref[...], preferred_element_type=jnp.float32)
        x = _leaky(z * nd_ref[...] + b_ref[...][None, :])
        hn = x * ns_ref[...]
        if last:
            w5_ref, o_ref = rest
            o_ref[...] = jnp.dot(hn, w5_ref[...],
                                 preferred_element_type=jnp.float32)
        else:
            (o_ref,) = rest
            for k in range(4):
                o_ref[k] = hn[:, 16 * k:16 * (k + 1)]

    in_specs = [
        pl.BlockSpec((4, Nb, 16), lambda i: (0, i, 0)),
        pl.BlockSpec((64, 64), lambda i: (0, 0)),
        pl.BlockSpec((64,), lambda i: (0,)),
        pl.BlockSpec((Nb, 1), lambda i: (i, 0)),
        pl.BlockSpec((Nb, 1), lambda i: (i, 0)),
    ]
    if last:
        in_specs.append(pl.BlockSpec((64, 3), lambda i: (0, 0)))
        out_specs = pl.BlockSpec((Nb, 3), lambda i: (i, 0))
        out_shape = jax.ShapeDtypeStruct((N_pad, 3), jnp.float32)
    else:
        out_specs = pl.BlockSpec((4, Nb, 16), lambda i: (0, i, 0))
        out_shape = jax.ShapeDtypeStruct((4, N_pad, 16), jnp.float32)
    return pl.pallas_call(
        body, grid=(N // Nb,), in_specs=in_specs, out_specs=out_specs,
        out_shape=out_shape)


def _tc_out(N, Nb):
    def body(p_ref, b_ref, nd_ref, o_ref):
        o_ref[...] = ((p_ref[0] + p_ref[1]) * nd_ref[...]
                      + b_ref[...][None, :])

    return pl.pallas_call(
        body,
        grid=(N // Nb,),
        in_specs=[
            pl.BlockSpec((2, Nb, 3), lambda i: (0, i, 0)),
            pl.BlockSpec((3,), lambda i: (0,)),
            pl.BlockSpec((Nb, 1), lambda i: (i, 0)),
        ],
        out_specs=pl.BlockSpec((Nb, 3), lambda i: (i, 0)),
        out_shape=jax.ShapeDtypeStruct((N, 3), jnp.float32),
    )


def kernel(features, edge_index, W1, b1, W2, b2, W3, b3, W4, b4, W5, b5):
    N = features.shape[0]
    E = edge_index.shape[1]
    assert N % 16 == 0
    N_pad = -(-(N + 1) // (NS * 8)) * (NS * 8)
    align = NC * NS * CHUNK * 2
    E_pad = -(-E // align) * align
    Nb = 4000 if N % 4000 == 0 else 1000
    assert N % Nb == 0

    src = edge_index[0]
    dst = edge_index[1]
    if E_pad != E:
        fill = jnp.full((E_pad - E,), N, jnp.int32)
        src = jnp.concatenate([src, fill])
        dst = jnp.concatenate([dst, fill])
    src2d = src.reshape(E_pad // SUB, SUB)
    dst2d = dst.reshape(E_pad // SUB, SUB)
    idx2d = jnp.concatenate([src, dst]).reshape(2 * E_pad // SUB, SUB)

    ones = jnp.ones((SUB, 1), jnp.float32)
    NpT = N_pad // NS
    z1 = jnp.zeros((NpT, 1), jnp.float32)
    z4 = jnp.zeros((NpT, 4), jnp.float32)
    z16 = jnp.zeros((NpT, 16), jnp.float32)
    z3 = jnp.zeros((NpT, 3), jnp.float32)

    deg2 = _sc_degree(N_pad, E_pad)(idx2d, ones, z1)
    ns_, nd_, h1 = _tc_norms(N, N_pad, Nb)(
        deg2[0, :N], deg2[1, :N], features)

    p1 = _sc_propagate_split(N_pad, E_pad, 4)(src2d, dst2d, h1, z4)
    h = _tc_layer1(N, N_pad, Nb)(p1, W1, b1, nd_, ns_)

    prop64 = _sc_propagate_sliced(N_pad, E_pad)
    for W, b in ((W2, b2), (W3, b3)):
        agg = prop64(src2d, dst2d, h, z16)
        h = _tc_mid(N, N_pad, Nb, last=False)(agg, W, b, nd_, ns_)

    agg = prop64(src2d, dst2d, h, z16)
    t5 = _tc_mid(N, N_pad, Nb, last=True)(agg, W4, b4, nd_, ns_, W5)

    p5 = _sc_propagate_split(N_pad, E_pad, 3)(src2d, dst2d, t5, z3)
    return _tc_out(N, Nb)(p5, b5, nd_)


# speedup vs baseline: 8.2583x; 1.0179x over previous
import functools

import jax
import jax.numpy as jnp
from jax import lax
from jax.experimental import pallas as pl
from jax.experimental.pallas import tpu as pltpu
from jax.experimental.pallas import tpu_sc as plsc

NC = 2
NS = 16
SUB = 512
KSUB = 1
CHUNK = SUB * KSUB


def _leaky(x):
    return jnp.where(x >= 0, x, 0.01 * x)


def _mesh():
    return plsc.VectorSubcoreMesh(core_axis_name="c", subcore_axis_name="s",
                                  num_cores=NC, num_subcores=NS)


def _run_pipeline(n_chunks, load_idx, wait_idx, do_gathers, fire_scatters,
                  drain_scatters, sync_scatters=False):
    assert n_chunks >= 4 and n_chunks % 2 == 0

    def iter_body(i, b, first, last):
        wait_idx(b)
        do_gathers(b)
        if sync_scatters:
            fire_scatters(b)
        else:
            if not first:
                drain_scatters(1 - b)
            fire_scatters(b)
        if not last:
            load_idx(i + 1, 1 - b)

    load_idx(0, 0)
    iter_body(0, 0, True, False)

    def pair(p, carry):
        i = 1 + 2 * p
        iter_body(i, 1, False, False)
        iter_body(i + 1, 0, False, False)
        return carry

    lax.fori_loop(0, (n_chunks - 2) // 2, pair, 0)
    iter_body(n_chunks - 1, 1, False, True)
    if not sync_scatters:
        drain_scatters(1)


def _sc_degree(N_pad, E_pad):
    K8 = 2
    NpT = N_pad // NS
    per_tile = E_pad // NS
    n_chunks = per_tile // (SUB * K8)
    rows_pt = per_tile // SUB
    rows_pc = E_pad // SUB
    assert per_tile % (SUB * K8) == 0

    @functools.partial(
        pl.kernel,
        out_type=jax.ShapeDtypeStruct((NC, N_pad, 1), jnp.float32),
        mesh=_mesh(),
        compiler_params=pltpu.CompilerParams(use_tc_tiling_on_sc=False),
        scratch_types=[
            pltpu.VMEM((K8, SUB), jnp.int32),
            pltpu.VMEM((SUB, 1), jnp.float32),
            pltpu.VMEM_SHARED((N_pad, 1), jnp.float32),
        ],
    )
    def kern(idx2d, ones_hbm, zeros_hbm, out_hbm, idxv, onesv, hist):
        c = lax.axis_index("c")
        t = lax.axis_index("s")
        pltpu.sync_copy(ones_hbm, onesv)
        pltpu.sync_copy(zeros_hbm, hist.at[pl.ds(t * NpT, NpT)])
        plsc.subcore_barrier()

        def step(i, carry):
            row0 = c * rows_pc + t * rows_pt + i * K8
            pltpu.sync_copy(idx2d.at[pl.ds(row0, K8)], idxv)
            for k in range(K8):
                pltpu.sync_copy(onesv, hist.at[idxv.at[k]], add=True)
            return carry

        lax.fori_loop(0, n_chunks, step, 0)
        plsc.subcore_barrier()

        @pl.when(c == 0)
        def _():
            pltpu.sync_copy(hist.at[pl.ds(t * NpT, NpT)],
                            out_hbm.at[0, pl.ds(t * NpT, NpT)])

        @pl.when(c == 1)
        def _():
            pltpu.sync_copy(hist.at[pl.ds(t * NpT, NpT)],
                            out_hbm.at[1, pl.ds(t * NpT, NpT)])

    return kern


def _sc_propagate_split(N_pad, E_pad, w):
    K8 = 2
    NpT = N_pad // NS
    per_tile = E_pad // (NC * NS)
    n_chunks = per_tile // (SUB * K8)
    rows_pt = per_tile // SUB
    rows_pc = E_pad // NC // SUB
    assert per_tile % (SUB * K8) == 0

    @functools.partial(
        pl.kernel,
        out_type=jax.ShapeDtypeStruct((NC, N_pad, w), jnp.float32),
        mesh=_mesh(),
        compiler_params=pltpu.CompilerParams(use_tc_tiling_on_sc=False),
        scratch_types=[
            pltpu.VMEM((K8, SUB), jnp.int32),
            pltpu.VMEM((K8, SUB), jnp.int32),
            pltpu.VMEM((SUB * K8, w), jnp.float32),
            pltpu.VMEM_SHARED((N_pad, w), jnp.float32),
            pltpu.SemaphoreType.DMA,
        ],
    )
    def kern(src2d, dst2d, h_hbm, zeros_hbm, out_hbm, idxs, idxd, rows, acc,
             sem):
        c = lax.axis_index("c")
        t = lax.axis_index("s")
        pltpu.sync_copy(zeros_hbm, acc.at[pl.ds(t * NpT, NpT)])
        plsc.subcore_barrier()

        def step(i, carry):
            row0 = c * rows_pc + t * rows_pt + i * K8
            pltpu.sync_copy(src2d.at[pl.ds(row0, K8)], idxs)
            pltpu.sync_copy(dst2d.at[pl.ds(row0, K8)], idxd)
            descs = [
                pltpu.async_copy(h_hbm.at[idxs.at[k]],
                                 rows.at[pl.ds(k * SUB, SUB)], sem)
                for k in range(K8)
            ]
            for d in descs:
                d.wait()
            for k in range(K8):
                pltpu.sync_copy(rows.at[pl.ds(k * SUB, SUB)],
                                acc.at[idxd.at[k]], add=True)
            return carry

        lax.fori_loop(0, n_chunks, step, 0)
        plsc.subcore_barrier()

        @pl.when(c == 0)
        def _():
            pltpu.sync_copy(acc.at[pl.ds(t * NpT, NpT)],
                            out_hbm.at[0, pl.ds(t * NpT, NpT)])

        @pl.when(c == 1)
        def _():
            pltpu.sync_copy(acc.at[pl.ds(t * NpT, NpT)],
                            out_hbm.at[1, pl.ds(t * NpT, NpT)])

    return kern


def _sc_propagate_sliced(N_pad, E_pad):
    NpT = N_pad // NS
    per_tile = E_pad // NS
    n_chunks = per_tile // CHUNK
    rows_pt = per_tile // SUB

    @functools.partial(
        pl.kernel,
        out_type=jax.ShapeDtypeStruct((4, N_pad, 16), jnp.float32),
        mesh=_mesh(),
        compiler_params=pltpu.CompilerParams(use_tc_tiling_on_sc=False),
        scratch_types=[
            pltpu.VMEM((2, KSUB, SUB), jnp.int32),
            pltpu.VMEM((2, KSUB, SUB), jnp.int32),
            pltpu.VMEM((2, CHUNK, 16), jnp.float32),
            pltpu.VMEM_SHARED((N_pad, 16), jnp.float32),
            pltpu.SemaphoreType.DMA,
            pltpu.SemaphoreType.DMA,
            pltpu.SemaphoreType.DMA,
            pltpu.SemaphoreType.DMA,
            pltpu.SemaphoreType.DMA,
            pltpu.SemaphoreType.DMA,
        ],
    )
    def kern(src2d, dst2d, h_hbm, zeros_hbm, out_hbm, idxs, idxd, rows, acc,
             si0, si1, sg0, sg1, ss0, ss1):
        c = lax.axis_index("c")
        t = lax.axis_index("s")
        sem_i = (si0, si1)
        sem_g = (sg0, sg1)
        sem_s = (ss0, ss1)

        def load_idx(i, b):
            row0 = t * rows_pt + i * KSUB
            pltpu.async_copy(src2d.at[pl.ds(row0, KSUB)], idxs.at[b], sem_i[b])
            pltpu.async_copy(dst2d.at[pl.ds(row0, KSUB)], idxd.at[b], sem_i[b])

        def wait_idx(b):
            pltpu.make_async_copy(src2d.at[pl.ds(0, KSUB)], idxs.at[b],
                                  sem_i[b]).wait()
            pltpu.make_async_copy(dst2d.at[pl.ds(0, KSUB)], idxd.at[b],
                                  sem_i[b]).wait()

        def fire_scatters(b):
            for k in range(KSUB):
                pltpu.async_copy(rows.at[b].at[pl.ds(k * SUB, SUB)],
                                 acc.at[idxd.at[b].at[k]], sem_s[b], add=True)

        def drain_scatters(b):
            for k in range(KSUB):
                pltpu.make_async_copy(rows.at[b].at[pl.ds(k * SUB, SUB)],
                                      acc.at[idxd.at[b].at[k]],
                                      sem_s[b]).wait()

        for j in range(2):

            def _fire_gathers(h2d, b):
                return [
                    pltpu.async_copy(h2d.at[idxs.at[b].at[k]],
                                     rows.at[b].at[pl.ds(k * SUB, SUB)],
                                     sem_g[b])
                    for k in range(KSUB)
                ]

            def do_gathers(b, j=j):
                @pl.when(c == 0)
                def _():
                    for d in _fire_gathers(h_hbm.at[j], b):
                        d.wait()

                @pl.when(c == 1)
                def _():
                    for d in _fire_gathers(h_hbm.at[2 + j], b):
                        d.wait()

            pltpu.sync_copy(zeros_hbm, acc.at[pl.ds(t * NpT, NpT)])
            plsc.subcore_barrier()
            _run_pipeline(n_chunks, load_idx, wait_idx, do_gathers,
                          fire_scatters, drain_scatters)
            plsc.subcore_barrier()

            @pl.when(c == 0)
            def _(j=j):
                pltpu.sync_copy(acc.at[pl.ds(t * NpT, NpT)],
                                out_hbm.at[j, pl.ds(t * NpT, NpT)])

            @pl.when(c == 1)
            def _(j=j):
                pltpu.sync_copy(acc.at[pl.ds(t * NpT, NpT)],
                                out_hbm.at[2 + j, pl.ds(t * NpT, NpT)])

    return kern


def _tc_norms(N, N_pad, Nb):
    def body(degs_ref, degd_ref, feat_ref, ns_ref, nd_ref, h1_ref):
        ns = lax.rsqrt(jnp.maximum(degs_ref[...], 1.0))
        nd = lax.rsqrt(jnp.maximum(degd_ref[...], 1.0))
        ns_ref[...] = ns
        nd_ref[...] = nd
        h1_ref[...] = feat_ref[...] * ns

    return pl.pallas_call(
        body,
        grid=(N // Nb,),
        in_specs=[
            pl.BlockSpec((Nb, 1), lambda i: (i, 0)),
            pl.BlockSpec((Nb, 1), lambda i: (i, 0)),
            pl.BlockSpec((Nb, 4), lambda i: (i, 0)),
        ],
        out_specs=[
            pl.BlockSpec((Nb, 1), lambda i: (i, 0)),
            pl.BlockSpec((Nb, 1), lambda i: (i, 0)),
            pl.BlockSpec((Nb, 4), lambda i: (i, 0)),
        ],
        out_shape=[
            jax.ShapeDtypeStruct((N, 1), jnp.float32),
            jax.ShapeDtypeStruct((N, 1), jnp.float32),
            jax.ShapeDtypeStruct((N_pad, 4), jnp.float32),
        ],
    )


def _tc_layer1(N, N_pad, Nb):
    def body(p_ref, w_ref, b_ref, nd_ref, ns_ref, o_ref):
        agg = p_ref[0] + p_ref[1]
        z = jnp.dot(agg, w_ref[...], preferred_element_type=jnp.float32)
        x = _leaky(z * nd_ref[...] + b_ref[...][None, :])
        hn = x * ns_ref[...]
        for k in range(4):
            o_ref[k] = hn[:, 16 * k:16 * (k + 1)]

    return pl.pallas_call(
        body,
        grid=(N // Nb,),
        in_specs=[
            pl.BlockSpec((2, Nb, 4), lambda i: (0, i, 0)),
            pl.BlockSpec((4, 64), lambda i: (0, 0)),
            pl.BlockSpec((64,), lambda i: (0,)),
            pl.BlockSpec((Nb, 1), lambda i: (i, 0)),
            pl.BlockSpec((Nb, 1), lambda i: (i, 0)),
        ],
        out_specs=pl.BlockSpec((4, Nb, 16), lambda i: (0, i, 0)),
        out_shape=jax.ShapeDtypeStruct((4, N_pad, 16), jnp.float32),
    )


def _tc_mid(N, N_pad, Nb, last):
    def body(a_ref, w_ref, b_ref, nd_ref, ns_ref, *rest):
        h = jnp.concatenate([a_ref[0], a_ref[1], a_ref[2], a_ref[3]], axis=1)
        z = jnp.dot(h, w_ref[...], preferred_element_type=jnp.float32)
        x = _leaky(z * nd_ref[...] + b_ref[...][None, :])
        hn = x * ns_ref[...]
        if last:
            w5_ref, o_ref = rest
            o_ref[...] = jnp.dot(hn, w5_ref[...],
                                 preferred_element_type=jnp.float32)
        else:
            (o_ref,) = rest
            for k in range(4):
                o_ref[k] = hn[:, 16 * k:16 * (k + 1)]

    in_specs = [
        pl.BlockSpec((4, Nb, 16), lambda i: (0, i, 0)),
        pl.BlockSpec((64, 64), lambda i: (0, 0)),
        pl.BlockSpec((64,), lambda i: (0,)),
        pl.BlockSpec((Nb, 1), lambda i: (i, 0)),
        pl.BlockSpec((Nb, 1), lambda i: (i, 0)),
    ]
    if last:
        in_specs.append(pl.BlockSpec((64, 3), lambda i: (0, 0)))
        out_specs = pl.BlockSpec((Nb, 3), lambda i: (i, 0))
        out_shape = jax.ShapeDtypeStruct((N_pad, 3), jnp.float32)
    else:
        out_specs = pl.BlockSpec((4, Nb, 16), lambda i: (0, i, 0))
        out_shape = jax.ShapeDtypeStruct((4, N_pad, 16), jnp.float32)
    return pl.pallas_call(
        body, grid=(N // Nb,), in_specs=in_specs, out_specs=out_specs,
        out_shape=out_shape)


def _tc_out(N, Nb):
    def body(p_ref, b_ref, nd_ref, o_ref):
        o_ref[...] = ((p_ref[0] + p_ref[1]) * nd_ref[...]
                      + b_ref[...][None, :])

    return pl.pallas_call(
        body,
        grid=(N // Nb,),
        in_specs=[
            pl.BlockSpec((2, Nb, 3), lambda i: (0, i, 0)),
            pl.BlockSpec((3,), lambda i: (0,)),
            pl.BlockSpec((Nb, 1), lambda i: (i, 0)),
        ],
        out_specs=pl.BlockSpec((Nb, 3), lambda i: (i, 0)),
        out_shape=jax.ShapeDtypeStruct((N, 3), jnp.float32),
    )


def kernel(features, edge_index, W1, b1, W2, b2, W3, b3, W4, b4, W5, b5):
    N = features.shape[0]
    E = edge_index.shape[1]
    assert N % 16 == 0
    N_pad = -(-(N + 1) // (NS * 8)) * (NS * 8)
    align = NC * NS * CHUNK * 2
    E_pad = -(-E // align) * align
    Nb = 4000 if N % 4000 == 0 else 1000
    assert N % Nb == 0

    src = edge_index[0]
    dst = edge_index[1]
    if E_pad != E:
        fill = jnp.full((E_pad - E,), N, jnp.int32)
        src = jnp.concatenate([src, fill])
        dst = jnp.concatenate([dst, fill])
    src2d = src.reshape(E_pad // SUB, SUB)
    dst2d = dst.reshape(E_pad // SUB, SUB)
    idx2d = jnp.concatenate([src, dst]).reshape(2 * E_pad // SUB, SUB)

    ones = jnp.ones((SUB, 1), jnp.float32)
    NpT = N_pad // NS
    z1 = jnp.zeros((NpT, 1), jnp.float32)
    z4 = jnp.zeros((NpT, 4), jnp.float32)
    z16 = jnp.zeros((NpT, 16), jnp.float32)
    z3 = jnp.zeros((NpT, 3), jnp.float32)

    deg2 = _sc_degree(N_pad, E_pad)(idx2d, ones, z1)
    ns_, nd_, h1 = _tc_norms(N, N_pad, Nb)(
        deg2[0, :N], deg2[1, :N], features)

    p1 = _sc_propagate_split(N_pad, E_pad, 4)(src2d, dst2d, h1, z4)
    h = _tc_layer1(N, N_pad, Nb)(p1, W1, b1, nd_, ns_)

    prop64 = _sc_propagate_sliced(N_pad, E_pad)
    for W, b in ((W2, b2), (W3, b3)):
        agg = prop64(src2d, dst2d, h, z16)
        h = _tc_mid(N, N_pad, Nb, last=False)(agg, W, b, nd_, ns_)

    agg = prop64(src2d, dst2d, h, z16)
    t5 = _tc_mid(N, N_pad, Nb, last=True)(agg, W4, b4, nd_, ns_, W5)

    p5 = _sc_propagate_split(N_pad, E_pad, 3)(src2d, dst2d, t5, z3)
    return _tc_out(N, Nb)(p5, b5, nd_)
